# baseline (speedup 1.0000x reference)
import functools

import jax
import jax.numpy as jnp
from jax import lax
from jax.experimental import pallas as pl
from jax.experimental.pallas import tpu as pltpu
from jax.experimental.pallas import tpu_sc as plsc

E = 16
K = 2
D = 2048
N = 4 * 4096
C = 2
NC = N // C
R = 1024
NT = NC // R
NW = 32
RS = NC // NW
L = 16
G = RS // L


def _tc_router(x_ref, w_ref, pin_ref, lt_ref, out2_ref, acc_ref, *, last):
    i = pl.program_id(0)
    lt = lax.dot_general(
        w_ref[...], x_ref[...],
        dimension_numbers=(((1,), (1,)), ((), ())),
        preferred_element_type=jnp.float32,
    )
    lt_ref[...] = lt
    m = jnp.max(lt, axis=0, keepdims=True)
    p = jnp.exp(lt - m)
    s = jnp.sum(p, axis=0, keepdims=True)
    part = jnp.sum(p / s, axis=1, keepdims=True)

    @pl.when(i == 0)
    def _():
        acc_ref[...] = jnp.zeros_like(acc_ref)

    acc_ref[...] += part

    @pl.when(i == pl.num_programs(0) - 1)
    def _():
        total = acc_ref[...] + pin_ref[...]
        if last:
            usage = total * (1.0 / N)
            out2_ref[...] = E * jnp.sum(
                usage * usage, axis=(0, 1), keepdims=True
            )
        else:
            out2_ref[...] = total


def _tc_call(xf, w, pin, c, last):
    body = functools.partial(_tc_router, last=last)
    out2 = (jax.ShapeDtypeStruct((1, 1), jnp.float32) if last
            else jax.ShapeDtypeStruct((E, 1), jnp.float32))
    out2_spec = (pl.BlockSpec((1, 1), lambda i: (0, 0)) if last
                 else pl.BlockSpec((E, 1), lambda i: (0, 0)))
    return pl.pallas_call(
        body,
        grid=(NT,),
        in_specs=[
            pl.BlockSpec((R, D), lambda i, c=c: (c * NT + i, 0)),
            pl.BlockSpec((E, D), lambda i: (0, 0)),
            pl.BlockSpec((E, 1), lambda i: (0, 0)),
        ],
        out_specs=[
            pl.BlockSpec((E, R), lambda i: (0, i)),
            out2_spec,
        ],
        out_shape=[
            jax.ShapeDtypeStruct((E, NC), jnp.float32),
            out2,
        ],
        scratch_shapes=[pltpu.VMEM((E, 1), jnp.float32)],
    )(xf, w, pin)


def _sc_router(lt_hbm, w_out, i_out, lt_v, w_v, i_v):
    nc = 2
    wid = lax.axis_index("s") * nc + lax.axis_index("c")
    base = wid * RS
    pltpu.sync_copy(lt_hbm.at[:, pl.ds(base, RS)], lt_v)

    def group(g, _):
        sl = pl.ds(g * L, L)
        m1 = lt_v[0, sl]
        i1 = jnp.zeros((L,), jnp.int32)
        m2 = jnp.full((L,), -3.0e38, jnp.float32)
        i2 = jnp.zeros((L,), jnp.int32)
        for e in range(1, E):
            le = lt_v[e, sl]
            es = jnp.full((L,), e, jnp.int32)
            gt1 = le > m1
            gt2 = le > m2
            n_m2 = jnp.where(gt1, m1, jnp.where(gt2, le, m2))
            n_i2 = jnp.where(gt1, i1, jnp.where(gt2, es, i2))
            m1 = jnp.where(gt1, le, m1)
            i1 = jnp.where(gt1, es, i1)
            m2 = n_m2
            i2 = n_i2
        e21 = jnp.exp(m2 - m1)
        den = 1.0 + e21
        w_v[0, sl] = 1.0 / den
        w_v[1, sl] = e21 / den
        i_v[0, sl] = i1
        i_v[1, sl] = i2
        return _

    lax.fori_loop(0, G, group, 0)

    pltpu.sync_copy(w_v, w_out.at[:, pl.ds(base, RS)])
    pltpu.sync_copy(i_v, i_out.at[:, pl.ds(base, RS)])


@functools.lru_cache(maxsize=1)
def _sc_call():
    return pl.kernel(
        _sc_router,
        mesh=plsc.VectorSubcoreMesh(core_axis_name="c", subcore_axis_name="s"),
        out_type=[
            jax.ShapeDtypeStruct((K, NC), jnp.float32),
            jax.ShapeDtypeStruct((K, NC), jnp.int32),
        ],
        scratch_types=[
            pltpu.VMEM((E, RS), jnp.float32),
            pltpu.VMEM((K, RS), jnp.float32),
            pltpu.VMEM((K, RS), jnp.int32),
        ],
    )


def kernel(x, W):
    b, t, d = x.shape
    xf = x.reshape(b * t, d)
    zeros = jnp.zeros((E, 1), jnp.float32)
    lt0, part0 = _tc_call(xf, W, zeros, 0, last=False)
    lt1, loss = _tc_call(xf, W, part0, 1, last=True)
    ws0, idx0 = _sc_call()(lt0)
    ws1, idx1 = _sc_call()(lt1)
    ws = jnp.concatenate((ws0, ws1), axis=1)
    idx = jnp.concatenate((idx0, idx1), axis=1)
    return (
        ws.T.reshape(b, t, K),
        idx.T.reshape(b, t, K),
        loss[0, 0],
    )
